# baseline (speedup 1.0000x reference)
import functools

import jax
import jax.numpy as jnp
from jax import lax
from jax.experimental import pallas as pl
from jax.experimental.pallas import tpu as pltpu
from jax.experimental.pallas import tpu_sc as plsc

BATCH = 16384
HIST = 50
HIST_PAD = 56
EMBED_DIM = 64


def _make_bagsum(num_workers):
    bags_per_w = BATCH // num_workers
    mesh = plsc.VectorSubcoreMesh(core_axis_name="c", subcore_axis_name="s")

    @functools.partial(
        pl.kernel,
        mesh=mesh,
        out_type=jax.ShapeDtypeStruct((BATCH, EMBED_DIM), jnp.float32),
        scratch_types=[
            pltpu.VMEM((bags_per_w, HIST_PAD), jnp.int32),
            pltpu.VMEM((HIST_PAD, EMBED_DIM), jnp.float32),
            pltpu.VMEM((bags_per_w, EMBED_DIM), jnp.float32),
            pltpu.SemaphoreType.DMA,
        ],
        compiler_params=pltpu.CompilerParams(use_tc_tiling_on_sc=False),
    )
    def bagsum(text_hbm, table_hbm, out_hbm, idx_v, rows_v, sums_v, sem):
        num_cores = jax.lax.axis_size("c")
        wid = lax.axis_index("s") * num_cores + lax.axis_index("c")
        base = wid * bags_per_w
        pltpu.sync_copy(text_hbm.at[pl.ds(base, bags_per_w)], idx_v)

        def body(b, _):
            pltpu.async_copy(table_hbm.at[idx_v.at[b]], rows_v, sem).wait()
            accs = []
            for g in range(EMBED_DIM // 16):
                acc = rows_v[0, pl.ds(g * 16, 16)]
                accs.append(acc)
            for j in range(1, HIST):
                for g in range(EMBED_DIM // 16):
                    accs[g] += rows_v[j, pl.ds(g * 16, 16)]
            for g in range(EMBED_DIM // 16):
                sums_v[b, pl.ds(g * 16, 16)] = accs[g]
            return 0

        lax.fori_loop(0, bags_per_w, body, 0)
        pltpu.sync_copy(sums_v, out_hbm.at[pl.ds(base, bags_per_w)])

    return bagsum


def _mlp_body(x_ref, w1_ref, b1_ref, w2_ref, b2_ref, o_ref):
    x = x_ref[...] * (1.0 / HIST)
    h = jnp.dot(x, w1_ref[...], preferred_element_type=jnp.float32) + b1_ref[...]
    o_ref[...] = jnp.dot(h, w2_ref[...], preferred_element_type=jnp.float32) + b2_ref[...]


def _mlp(sums, fc1_w, fc1_b, fc2_w, fc2_b):
    blk = 2048
    grid = (BATCH // blk,)
    nclass = fc2_w.shape[0]
    hid = fc1_w.shape[0]
    return pl.pallas_call(
        _mlp_body,
        grid=grid,
        in_specs=[
            pl.BlockSpec((blk, EMBED_DIM), lambda i: (i, 0)),
            pl.BlockSpec((EMBED_DIM, hid), lambda i: (0, 0)),
            pl.BlockSpec((1, hid), lambda i: (0, 0)),
            pl.BlockSpec((hid, nclass), lambda i: (0, 0)),
            pl.BlockSpec((1, nclass), lambda i: (0, 0)),
        ],
        out_specs=pl.BlockSpec((blk, nclass), lambda i: (i, 0)),
        out_shape=jax.ShapeDtypeStruct((BATCH, nclass), jnp.float32),
    )(sums, fc1_w.T, fc1_b.reshape(1, hid), fc2_w.T, fc2_b.reshape(1, nclass))


def kernel(text, emb_weight, fc1_w, fc1_b, fc2_w, fc2_b):
    text = text.astype(jnp.int32)
    pad = jnp.zeros((BATCH, HIST_PAD - HIST), dtype=jnp.int32)
    text_pad = jnp.concatenate([text, pad], axis=1)
    sums = _make_bagsum(32)(text_pad, emb_weight)
    return _mlp(sums, fc1_w, fc1_b, fc2_w, fc2_b)


# speedup vs baseline: 1.8910x; 1.8910x over previous
import functools

import jax
import jax.numpy as jnp
from jax import lax
from jax.experimental import pallas as pl
from jax.experimental.pallas import tpu as pltpu
from jax.experimental.pallas import tpu_sc as plsc

BATCH = 16384
HIST = 50
PAIR = 2 * HIST + 4
EMBED_DIM = 64
NG = EMBED_DIM // 16

NUM_WORKERS = 32
BAGS_PER_W = BATCH // NUM_WORKERS
PAIRS_PER_W = BAGS_PER_W // 2
K = 4
GROUPS = PAIRS_PER_W // K
DMA_BYTES = PAIR * EMBED_DIM * 4


def _make_bagsum():
    mesh = plsc.VectorSubcoreMesh(core_axis_name="c", subcore_axis_name="s")

    @functools.partial(
        pl.kernel,
        mesh=mesh,
        out_type=jax.ShapeDtypeStruct((BATCH, EMBED_DIM), jnp.float32),
        scratch_types=[
            pltpu.VMEM((PAIRS_PER_W, PAIR), jnp.int32),
            pltpu.VMEM((2, K, PAIR, EMBED_DIM), jnp.float32),
            pltpu.VMEM((BAGS_PER_W, EMBED_DIM), jnp.float32),
            pltpu.SemaphoreType.DMA,
        ],
        compiler_params=pltpu.CompilerParams(use_tc_tiling_on_sc=False),
    )
    def bagsum(text_hbm, table_hbm, out_hbm, idx_v, rows_v, sums_v, sem):
        num_cores = jax.lax.axis_size("c")
        wid = lax.axis_index("s") * num_cores + lax.axis_index("c")
        pltpu.sync_copy(text_hbm.at[pl.ds(wid * PAIRS_PER_W, PAIRS_PER_W)], idx_v)

        def fire_group(g, p):
            for j in range(K):
                pltpu.async_copy(
                    table_hbm.at[idx_v.at[g * K + j]], rows_v.at[p, j], sem
                )

        def drain_group(p):
            for j in range(K):
                pltpu.make_async_copy(
                    table_hbm.at[idx_v.at[0]], rows_v.at[p, j], sem
                ).wait()

        def sum_group(g, p):
            for j in range(K):
                buf = rows_v.at[p, j]
                for half in range(2):
                    base = half * HIST
                    accs = [buf[base, pl.ds(gg * 16, 16)] for gg in range(NG)]

                    def row_body(it, accs, _base=base, _buf=buf):
                        r = _base + 1 + it * 7
                        new = list(accs)
                        for u in range(7):
                            for gg in range(NG):
                                new[gg] += _buf[r + u, pl.ds(gg * 16, 16)]
                        return tuple(new)

                    accs = lax.fori_loop(0, 7, row_body, tuple(accs))
                    bag = (g * K + j) * 2 + half
                    for gg in range(NG):
                        sums_v[bag, pl.ds(gg * 16, 16)] = accs[gg]

        fire_group(0, 0)

        def outer(i, _):
            g0 = 2 * i
            fire_group(g0 + 1, 1)
            drain_group(0)
            sum_group(g0, 0)
            g2 = lax.rem(g0 + 2, GROUPS)
            fire_group(g2, 0)
            drain_group(1)
            sum_group(g0 + 1, 1)
            return 0

        lax.fori_loop(0, GROUPS // 2, outer, 0)
        drain_group(0)
        pltpu.sync_copy(sums_v, out_hbm.at[pl.ds(wid * BAGS_PER_W, BAGS_PER_W)])

    return bagsum


def _mlp_body(x_ref, w1_ref, b1_ref, w2_ref, b2_ref, o_ref):
    x = x_ref[...] * (1.0 / HIST)
    h = jnp.dot(x, w1_ref[...], preferred_element_type=jnp.float32) + b1_ref[...]
    o_ref[...] = jnp.dot(h, w2_ref[...], preferred_element_type=jnp.float32) + b2_ref[...]


def _mlp(sums, fc1_w, fc1_b, fc2_w, fc2_b):
    blk = 2048
    grid = (BATCH // blk,)
    nclass = fc2_w.shape[0]
    hid = fc1_w.shape[0]
    return pl.pallas_call(
        _mlp_body,
        grid=grid,
        in_specs=[
            pl.BlockSpec((blk, EMBED_DIM), lambda i: (i, 0)),
            pl.BlockSpec((EMBED_DIM, hid), lambda i: (0, 0)),
            pl.BlockSpec((1, hid), lambda i: (0, 0)),
            pl.BlockSpec((hid, nclass), lambda i: (0, 0)),
            pl.BlockSpec((1, nclass), lambda i: (0, 0)),
        ],
        out_specs=pl.BlockSpec((blk, nclass), lambda i: (i, 0)),
        out_shape=jax.ShapeDtypeStruct((BATCH, nclass), jnp.float32),
    )(sums, fc1_w.T, fc1_b.reshape(1, hid), fc2_w.T, fc2_b.reshape(1, nclass))


def kernel(text, emb_weight, fc1_w, fc1_b, fc2_w, fc2_b):
    text = text.astype(jnp.int32)
    pairs = text.reshape(BATCH // 2, 2 * HIST)
    pad = jnp.zeros((BATCH // 2, PAIR - 2 * HIST), dtype=jnp.int32)
    idx = jnp.concatenate([pairs, pad], axis=1)
    sums = _make_bagsum()(idx, emb_weight)
    return _mlp(sums, fc1_w, fc1_b, fc2_w, fc2_b)


# speedup vs baseline: 3.5323x; 1.8680x over previous
import functools

import jax
import jax.numpy as jnp
from jax import lax
from jax.experimental import pallas as pl
from jax.experimental.pallas import tpu as pltpu
from jax.experimental.pallas import tpu_sc as plsc

BATCH = 16384
HIST = 50
PAIR = 2 * HIST + 4
EMBED_DIM = 64
NG = EMBED_DIM // 16

NUM_WORKERS = 32
BAGS_PER_W = BATCH // NUM_WORKERS
PAIRS_PER_W = BAGS_PER_W // 2
K = 4
GROUPS = PAIRS_PER_W // K
DMA_BYTES = PAIR * EMBED_DIM * 4


def _make_bagsum():
    mesh = plsc.VectorSubcoreMesh(core_axis_name="c", subcore_axis_name="s")

    @functools.partial(
        pl.kernel,
        mesh=mesh,
        out_type=jax.ShapeDtypeStruct((BATCH, EMBED_DIM), jnp.float32),
        scratch_types=[
            pltpu.VMEM((PAIRS_PER_W, PAIR), jnp.int32),
            pltpu.VMEM((2, K, PAIR, EMBED_DIM), jnp.float32),
            pltpu.VMEM((BAGS_PER_W, EMBED_DIM), jnp.float32),
            pltpu.SemaphoreType.DMA,
        ],
        compiler_params=pltpu.CompilerParams(use_tc_tiling_on_sc=False),
    )
    def bagsum(text_hbm, table_hbm, out_hbm, idx_v, rows_v, sums_v, sem):
        num_cores = jax.lax.axis_size("c")
        wid = lax.axis_index("s") * num_cores + lax.axis_index("c")
        pltpu.sync_copy(text_hbm.at[pl.ds(wid * PAIRS_PER_W, PAIRS_PER_W)], idx_v)

        def fire_group(g, p):
            for j in range(K):
                pltpu.async_copy(
                    table_hbm.at[idx_v.at[g * K + j]], rows_v.at[p, j], sem
                )

        def drain_group(p):
            for j in range(K):
                pltpu.make_async_copy(
                    table_hbm.at[idx_v.at[0]], rows_v.at[p, j], sem
                ).wait()

        def sum_group(g, p):
            for j in range(K):
                buf = rows_v.at[p, j]
                for half in range(2):
                    base = half * HIST
                    accs = [buf[base, pl.ds(gg * 16, 16)] for gg in range(NG)]

                    def row_body(it, accs, _base=base, _buf=buf):
                        r = _base + 1 + it * 7
                        new = list(accs)
                        for u in range(7):
                            for gg in range(NG):
                                new[gg] += _buf[r + u, pl.ds(gg * 16, 16)]
                        return tuple(new)

                    accs = lax.fori_loop(0, 7, row_body, tuple(accs))
                    bag = (g * K + j) * 2 + half
                    for gg in range(NG):
                        sums_v[bag, pl.ds(gg * 16, 16)] = accs[gg]

        fire_group(0, 0)

        def outer(i, _):
            g0 = 2 * i
            fire_group(g0 + 1, 1)
            drain_group(0)
            sum_group(g0, 0)
            g2 = lax.rem(g0 + 2, GROUPS)
            fire_group(g2, 0)
            drain_group(1)
            sum_group(g0 + 1, 1)
            return 0

        lax.fori_loop(0, GROUPS // 2, outer, 0)
        drain_group(0)
        pltpu.sync_copy(sums_v, out_hbm.at[pl.ds(wid * BAGS_PER_W, BAGS_PER_W)])

    return bagsum


def _mlp_body(x_ref, w1_ref, b1_ref, w2_ref, b2_ref, o_ref):
    x = x_ref[...] * (1.0 / HIST)
    h = jnp.dot(x, w1_ref[...], preferred_element_type=jnp.float32) + b1_ref[...]
    o_ref[...] = jnp.dot(h, w2_ref[...], preferred_element_type=jnp.float32) + b2_ref[...]


def _mlp(sums, fc1_w, fc1_b, fc2_w, fc2_b):
    blk = 2048
    grid = (BATCH // blk,)
    nclass = fc2_w.shape[0]
    hid = fc1_w.shape[0]
    return pl.pallas_call(
        _mlp_body,
        grid=grid,
        in_specs=[
            pl.BlockSpec((blk, EMBED_DIM), lambda i: (i, 0)),
            pl.BlockSpec((EMBED_DIM, hid), lambda i: (0, 0)),
            pl.BlockSpec((1, hid), lambda i: (0, 0)),
            pl.BlockSpec((hid, nclass), lambda i: (0, 0)),
            pl.BlockSpec((1, nclass), lambda i: (0, 0)),
        ],
        out_specs=pl.BlockSpec((blk, nclass), lambda i: (i, 0)),
        out_shape=jax.ShapeDtypeStruct((BATCH, nclass), jnp.float32),
    )(sums, fc1_w.T, fc1_b.reshape(1, hid), fc2_w.T, fc2_b.reshape(1, nclass))


def kernel(text, emb_weight, fc1_w, fc1_b, fc2_w, fc2_b):
    text = text.astype(jnp.int32)
    pairs = text.reshape(BATCH // 2, 2 * HIST)
    npad = PAIR - 2 * HIST
    pad = (jnp.arange(BATCH // 2, dtype=jnp.int32)[:, None] * npad
           + jnp.arange(npad, dtype=jnp.int32)[None, :])
    idx = jnp.concatenate([pairs, pad], axis=1)
    sums = _make_bagsum()(idx, emb_weight)
    return _mlp(sums, fc1_w, fc1_b, fc2_w, fc2_b)


# speedup vs baseline: 5.4761x; 1.5503x over previous
import functools

import jax
import jax.numpy as jnp
from jax import lax
from jax.experimental import pallas as pl
from jax.experimental.pallas import tpu as pltpu
from jax.experimental.pallas import tpu_sc as plsc

VOCAB = 1000000
EMBED_DIM = 64
NCLS = 10
CPAD = 16
BATCH = 16384
HIST = 50
PAIR = 2 * HIST + 4

R = 1024
SEG = 124928
NK = 8
MAIN = NK * SEG
TAIL = VOCAB - MAIN
OUTROWS = SEG + TAIL
GROWS = OUTROWS * 8

NUM_WORKERS = 32
BAGS_PER_W = BATCH // NUM_WORKERS
PAIRS_PER_W = BAGS_PER_W // 2
K = 8
GROUPS = PAIRS_PER_W // K


def _proj_body(*refs):
    et_refs = refs[:NK]
    et_tail_ref, w1t_ref, w2t_ref, o_ref = refs[NK:]
    i = pl.program_id(0)
    w = jnp.dot(w1t_ref[...], w2t_ref[...],
                preferred_element_type=jnp.float32) * (1.0 / HIST)

    @pl.when(i < SEG // R)
    def _main():
        ps = []
        for k in range(NK):
            ps.append(lax.dot_general(et_refs[k][...], w, (((0,), (0,)), ((), ())),
                                      preferred_element_type=jnp.float32))
        o_ref[...] = jnp.concatenate(ps, axis=1)

    @pl.when(i == SEG // R)
    def _tail():
        p = lax.dot_general(et_tail_ref[...], w, (((0,), (0,)), ((), ())),
                            preferred_element_type=jnp.float32)
        o_ref[0:TAIL, :] = jnp.concatenate([p] * NK, axis=1)


def _project(et, et_tail, w1t, w2t):
    nblk = SEG // R
    ins = [pl.BlockSpec((EMBED_DIM, R),
                        (lambda k: (lambda i: (0, jnp.minimum(i, nblk - 1) + k * nblk)))(k))
           for k in range(NK)]
    ins += [pl.BlockSpec((EMBED_DIM, TAIL), lambda i: (0, 0)),
            pl.BlockSpec((EMBED_DIM, 32), lambda i: (0, 0)),
            pl.BlockSpec((32, CPAD), lambda i: (0, 0))]
    return pl.pallas_call(
        _proj_body, grid=(nblk + 1,), in_specs=ins,
        out_specs=pl.BlockSpec((R, NK * CPAD), lambda i: (i, 0)),
        out_shape=jax.ShapeDtypeStruct((OUTROWS, NK * CPAD), jnp.float32),
    )(*([et] * NK), et_tail, w1t, w2t)


def _make_bagsum():
    mesh = plsc.VectorSubcoreMesh(core_axis_name="c", subcore_axis_name="s")

    @functools.partial(
        pl.kernel,
        mesh=mesh,
        out_type=jax.ShapeDtypeStruct((BATCH, CPAD), jnp.float32),
        scratch_types=[
            pltpu.VMEM((PAIRS_PER_W, PAIR), jnp.int32),
            pltpu.VMEM((2, K, PAIR, CPAD), jnp.float32),
            pltpu.VMEM((BAGS_PER_W, CPAD), jnp.float32),
            pltpu.SemaphoreType.DMA,
        ],
        compiler_params=pltpu.CompilerParams(use_tc_tiling_on_sc=False),
    )
    def bagsum(text_hbm, table_hbm, out_hbm, idx_v, rows_v, sums_v, sem):
        num_cores = jax.lax.axis_size("c")
        wid = lax.axis_index("s") * num_cores + lax.axis_index("c")
        pltpu.sync_copy(text_hbm.at[pl.ds(wid * PAIRS_PER_W, PAIRS_PER_W)], idx_v)

        def fire_group(g, p):
            for j in range(K):
                pltpu.async_copy(
                    table_hbm.at[idx_v.at[g * K + j]], rows_v.at[p, j], sem
                )

        def drain_group(p):
            for j in range(K):
                pltpu.make_async_copy(
                    table_hbm.at[idx_v.at[0]], rows_v.at[p, j], sem
                ).wait()

        def sum_group(g, p):
            for j in range(K):
                buf = rows_v.at[p, j]
                for half in range(2):
                    base = half * HIST
                    acc = buf[base, :]

                    def row_body(it, acc, _base=base, _buf=buf):
                        r = _base + 1 + it * 7
                        for u in range(7):
                            acc += _buf[r + u, :]
                        return acc

                    acc = lax.fori_loop(0, 7, row_body, acc)
                    sums_v[(g * K + j) * 2 + half, :] = acc

        fire_group(0, 0)

        def outer(i, _):
            g0 = 2 * i
            fire_group(g0 + 1, 1)
            drain_group(0)
            sum_group(g0, 0)
            g2 = lax.rem(g0 + 2, GROUPS)
            fire_group(g2, 0)
            drain_group(1)
            sum_group(g0 + 1, 1)
            return 0

        lax.fori_loop(0, GROUPS // 2, outer, 0)
        drain_group(0)
        pltpu.sync_copy(sums_v, out_hbm.at[pl.ds(wid * BAGS_PER_W, BAGS_PER_W)])

    return bagsum


def _epi_body(x_ref, b1_ref, w2t_ref, b2_ref, o_ref):
    bias = jnp.dot(b1_ref[...], w2t_ref[...],
                   preferred_element_type=jnp.float32) + b2_ref[...]
    o_ref[...] = x_ref[:, :NCLS] + bias


def _epilogue(sums, fc1_b, fc2_w, fc2_b):
    blk = 2048
    return pl.pallas_call(
        _epi_body,
        grid=(BATCH // blk,),
        in_specs=[
            pl.BlockSpec((blk, CPAD), lambda i: (i, 0)),
            pl.BlockSpec((1, 32), lambda i: (0, 0)),
            pl.BlockSpec((32, NCLS), lambda i: (0, 0)),
            pl.BlockSpec((1, NCLS), lambda i: (0, 0)),
        ],
        out_specs=pl.BlockSpec((blk, NCLS), lambda i: (i, 0)),
        out_shape=jax.ShapeDtypeStruct((BATCH, NCLS), jnp.float32),
    )(sums, fc1_b.reshape(1, 32), fc2_w.T, fc2_b.reshape(1, NCLS))


def kernel(text, emb_weight, fc1_w, fc1_b, fc2_w, fc2_b):
    et = emb_weight.T
    et_tail = lax.slice(et, (0, MAIN), (EMBED_DIM, VOCAB))
    w2t = jnp.pad(fc2_w.T, ((0, 0), (0, CPAD - NCLS)))
    p = _project(et, et_tail, fc1_w.T, w2t)
    table16 = p.reshape(GROWS, CPAD)

    v = text.astype(jnp.int32)
    linrow = jnp.where(v < MAIN, (v % SEG) * 8 + v // SEG, (SEG + (v - MAIN)) * 8)
    pairs = linrow.reshape(BATCH // 2, 2 * HIST)
    npad = PAIR - 2 * HIST
    pad = (jnp.arange(BATCH // 2, dtype=jnp.int32)[:, None] * npad
           + jnp.arange(npad, dtype=jnp.int32)[None, :]) * 8
    idx = jnp.concatenate([pairs, pad], axis=1)

    sums = _make_bagsum()(idx, table16)
    return _epilogue(sums, fc1_b, fc2_w, fc2_b)


# speedup vs baseline: 7.9303x; 1.4482x over previous
import functools

import jax
import jax.numpy as jnp
from jax import lax
from jax.experimental import pallas as pl
from jax.experimental.pallas import tpu as pltpu
from jax.experimental.pallas import tpu_sc as plsc

VOCAB = 1000000
EMBED_DIM = 64
NCLS = 10
CPAD = 16
BATCH = 16384
HIST = 50
PAIR = 2 * HIST + 4

R = 1024
SEG = 124928
NK = 8
MAIN = NK * SEG
TAIL = VOCAB - MAIN
OUTROWS = SEG + TAIL
GROWS = OUTROWS * 8

NUM_WORKERS = 32
BAGS_PER_W = BATCH // NUM_WORKERS
PAIRS_PER_W = BAGS_PER_W // 2
K = 8
GROUPS = PAIRS_PER_W // K


def _proj_body(*refs):
    et_refs = refs[:NK]
    et_tail_ref, w1t_ref, w2t_ref, o_ref = refs[NK:]
    i = pl.program_id(0)
    wt = lax.dot_general(w2t_ref[...], w1t_ref[...], (((0,), (1,)), ((), ())),
                         preferred_element_type=jnp.float32) * (1.0 / HIST)

    @pl.when(i < SEG // R)
    def _main():
        ps = []
        for k in range(NK):
            ps.append(jnp.dot(wt, et_refs[k][...], preferred_element_type=jnp.float32))
        o_ref[...] = jnp.concatenate(ps, axis=0).T

    @pl.when(i == SEG // R)
    def _tail():
        pt = jnp.dot(wt, et_tail_ref[...], preferred_element_type=jnp.float32)
        o_ref[0:TAIL, :] = jnp.concatenate([pt] * NK, axis=0).T


def _project(et, et_tail, w1t, w2t):
    nblk = SEG // R
    ins = [pl.BlockSpec((EMBED_DIM, R),
                        (lambda k: (lambda i: (0, jnp.minimum(i, nblk - 1) + k * nblk)))(k))
           for k in range(NK)]
    ins += [pl.BlockSpec((EMBED_DIM, TAIL), lambda i: (0, 0)),
            pl.BlockSpec((EMBED_DIM, 32), lambda i: (0, 0)),
            pl.BlockSpec((32, CPAD), lambda i: (0, 0))]
    return pl.pallas_call(
        _proj_body, grid=(nblk + 1,), in_specs=ins,
        out_specs=pl.BlockSpec((R, NK * CPAD), lambda i: (i, 0)),
        out_shape=jax.ShapeDtypeStruct((OUTROWS, NK * CPAD), jnp.float32),
        compiler_params=pltpu.CompilerParams(fuse_transposed_lhs_in_matmul=True),
    )(*([et] * NK), et_tail, w1t, w2t)


def _make_bagsum():
    mesh = plsc.VectorSubcoreMesh(core_axis_name="c", subcore_axis_name="s")

    @functools.partial(
        pl.kernel,
        mesh=mesh,
        out_type=jax.ShapeDtypeStruct((BATCH, CPAD), jnp.float32),
        scratch_types=[
            pltpu.VMEM((PAIRS_PER_W, PAIR), jnp.int32),
            pltpu.VMEM((2, K, PAIR, CPAD), jnp.float32),
            pltpu.VMEM((BAGS_PER_W, CPAD), jnp.float32),
            pltpu.SemaphoreType.DMA,
        ],
        compiler_params=pltpu.CompilerParams(use_tc_tiling_on_sc=False),
    )
    def bagsum(text_hbm, table_hbm, out_hbm, idx_v, rows_v, sums_v, sem):
        num_cores = jax.lax.axis_size("c")
        wid = lax.axis_index("s") * num_cores + lax.axis_index("c")
        pltpu.sync_copy(text_hbm.at[pl.ds(wid * PAIRS_PER_W, PAIRS_PER_W)], idx_v)

        def fire_group(g, p):
            for j in range(K):
                pltpu.async_copy(
                    table_hbm.at[idx_v.at[g * K + j]], rows_v.at[p, j], sem
                )

        def drain_group(p):
            for j in range(K):
                pltpu.make_async_copy(
                    table_hbm.at[idx_v.at[0]], rows_v.at[p, j], sem
                ).wait()

        def sum_group(g, p):
            for j in range(K):
                buf = rows_v.at[p, j]
                for half in range(2):
                    base = half * HIST
                    acc = buf[base, :]

                    def row_body(it, acc, _base=base, _buf=buf):
                        r = _base + 1 + it * 7
                        for u in range(7):
                            acc += _buf[r + u, :]
                        return acc

                    acc = lax.fori_loop(0, 7, row_body, acc)
                    sums_v[(g * K + j) * 2 + half, :] = acc

        fire_group(0, 0)

        def outer(i, _):
            g0 = 2 * i
            fire_group(g0 + 1, 1)
            drain_group(0)
            sum_group(g0, 0)
            g2 = lax.rem(g0 + 2, GROUPS)
            fire_group(g2, 0)
            drain_group(1)
            sum_group(g0 + 1, 1)
            return 0

        lax.fori_loop(0, GROUPS // 2, outer, 0)
        drain_group(0)
        pltpu.sync_copy(sums_v, out_hbm.at[pl.ds(wid * BAGS_PER_W, BAGS_PER_W)])

    return bagsum


def _epi_body(x_ref, b1_ref, w2t_ref, b2_ref, o_ref):
    bias = jnp.dot(b1_ref[...], w2t_ref[...],
                   preferred_element_type=jnp.float32) + b2_ref[...]
    o_ref[...] = x_ref[:, :NCLS] + bias


def _epilogue(sums, fc1_b, fc2_w, fc2_b):
    blk = 2048
    return pl.pallas_call(
        _epi_body,
        grid=(BATCH // blk,),
        in_specs=[
            pl.BlockSpec((blk, CPAD), lambda i: (i, 0)),
            pl.BlockSpec((1, 32), lambda i: (0, 0)),
            pl.BlockSpec((32, NCLS), lambda i: (0, 0)),
            pl.BlockSpec((1, NCLS), lambda i: (0, 0)),
        ],
        out_specs=pl.BlockSpec((blk, NCLS), lambda i: (i, 0)),
        out_shape=jax.ShapeDtypeStruct((BATCH, NCLS), jnp.float32),
    )(sums, fc1_b.reshape(1, 32), fc2_w.T, fc2_b.reshape(1, NCLS))


def kernel(text, emb_weight, fc1_w, fc1_b, fc2_w, fc2_b):
    et = emb_weight.T
    et_tail = lax.slice(et, (0, MAIN), (EMBED_DIM, VOCAB))
    w2t = jnp.pad(fc2_w.T, ((0, 0), (0, CPAD - NCLS)))
    p = _project(et, et_tail, fc1_w.T, w2t)
    table16 = p.reshape(GROWS, CPAD)

    v = text.astype(jnp.int32)
    linrow = jnp.where(v < MAIN, (v % SEG) * 8 + v // SEG, (SEG + (v - MAIN)) * 8)
    pairs = linrow.reshape(BATCH // 2, 2 * HIST)
    npad = PAIR - 2 * HIST
    pad = (jnp.arange(BATCH // 2, dtype=jnp.int32)[:, None] * npad
           + jnp.arange(npad, dtype=jnp.int32)[None, :]) * 8
    idx = jnp.concatenate([pairs, pad], axis=1)

    sums = _make_bagsum()(idx, table16)
    return _epilogue(sums, fc1_b, fc2_w, fc2_b)


# speedup vs baseline: 8.3886x; 1.0578x over previous
import functools

import jax
import jax.numpy as jnp
from jax import lax
from jax.experimental import pallas as pl
from jax.experimental.pallas import tpu as pltpu
from jax.experimental.pallas import tpu_sc as plsc

VOCAB = 1000000
EMBED_DIM = 64
NCLS = 10
CPAD = 16
BATCH = 16384
HIST = 50
PAIR = 2 * HIST + 4

R = 1024
NK = 8
STEP = NK * R
NSTEPS = VOCAB // STEP
MAIN = NSTEPS * STEP
TAIL = VOCAB - MAIN
OUTROWS = NSTEPS * R + TAIL + 1
BIASROW = (NSTEPS * R + TAIL) * 8
GROWS = OUTROWS * 8

NUM_WORKERS = 32
BAGS_PER_W = BATCH // NUM_WORKERS
PAIRS_PER_W = BAGS_PER_W // 2
IDX_PER_W = PAIRS_PER_W * PAIR
K = 8
GROUPS = PAIRS_PER_W // K


def _proj_body(*refs):
    et_refs = refs[:NK]
    et_tail_ref, w1t_ref, w2t_ref, b1_ref, b2_ref, o_ref = refs[NK:]
    i = pl.program_id(0)
    wt = lax.dot_general(w2t_ref[...], w1t_ref[...], (((0,), (1,)), ((), ())),
                         preferred_element_type=jnp.float32) * (1.0 / HIST)

    @pl.when(i < NSTEPS)
    def _main():
        ps = []
        for k in range(NK):
            ps.append(jnp.dot(wt, et_refs[k][...], preferred_element_type=jnp.float32))
        o_ref[...] = jnp.concatenate(ps, axis=0).T

    @pl.when(i == NSTEPS)
    def _tail():
        pt = jnp.dot(wt, et_tail_ref[...], preferred_element_type=jnp.float32)
        o_ref[0:TAIL, :] = jnp.concatenate([pt] * NK, axis=0).T
        o_ref[TAIL:TAIL + 1, 0:CPAD] = (
            jnp.dot(b1_ref[...], w2t_ref[...], preferred_element_type=jnp.float32)
            + b2_ref[...]
        )


def _project(et, et_tail, w1t, w2t, b1, b2):
    ins = [pl.BlockSpec((EMBED_DIM, R), (lambda k: (lambda i: (0, jnp.minimum(i * NK + k, NSTEPS * NK - 1))))(k))
           for k in range(NK)]
    ins += [pl.BlockSpec((EMBED_DIM, TAIL), lambda i: (0, 0)),
            pl.BlockSpec((EMBED_DIM, 32), lambda i: (0, 0)),
            pl.BlockSpec((32, CPAD), lambda i: (0, 0)),
            pl.BlockSpec((1, 32), lambda i: (0, 0)),
            pl.BlockSpec((1, CPAD), lambda i: (0, 0))]
    return pl.pallas_call(
        _proj_body, grid=(NSTEPS + 1,), in_specs=ins,
        out_specs=pl.BlockSpec((R, NK * CPAD), lambda i: (i, 0)),
        out_shape=jax.ShapeDtypeStruct((OUTROWS, NK * CPAD), jnp.float32),
    )(*([et] * NK), et_tail, w1t, w2t, b1, b2)


def _make_bagsum():
    mesh = plsc.VectorSubcoreMesh(core_axis_name="c", subcore_axis_name="s")

    @functools.partial(
        pl.kernel,
        mesh=mesh,
        out_type=jax.ShapeDtypeStruct((BATCH, CPAD), jnp.float32),
        scratch_types=[
            pltpu.VMEM((IDX_PER_W,), jnp.int32),
            pltpu.VMEM((2, K, 2 * HIST, CPAD), jnp.float32),
            pltpu.VMEM((BAGS_PER_W, CPAD), jnp.float32),
            pltpu.VMEM((8, CPAD), jnp.float32),
            pltpu.SemaphoreType.DMA,
        ],
        compiler_params=pltpu.CompilerParams(use_tc_tiling_on_sc=False),
    )
    def bagsum(idx_hbm, table_hbm, out_hbm, idx_v, rows_v, sums_v, bias_v, sem):
        num_cores = jax.lax.axis_size("c")
        wid = lax.axis_index("s") * num_cores + lax.axis_index("c")
        pltpu.sync_copy(idx_hbm.at[pl.ds(wid * IDX_PER_W, IDX_PER_W)], idx_v)
        pltpu.sync_copy(table_hbm.at[pl.ds(BIASROW, 8)], bias_v)

        def fire_group(g, p):
            for j in range(K):
                pltpu.async_copy(
                    table_hbm.at[idx_v.at[pl.ds((g * K + j) * PAIR, 2 * HIST)]],
                    rows_v.at[p, j], sem,
                )

        def drain_group(p):
            for j in range(K):
                pltpu.make_async_copy(
                    table_hbm.at[idx_v.at[pl.ds(0, 2 * HIST)]],
                    rows_v.at[p, j], sem,
                ).wait()

        def sum_group(g, p):
            bias = bias_v[0, :]
            for j in range(K):
                buf = rows_v.at[p, j]
                for half in range(2):
                    base = half * HIST
                    acc = buf[base, :]

                    def row_body(it, acc, _base=base, _buf=buf):
                        r = _base + 1 + it * 7
                        for u in range(7):
                            acc += _buf[r + u, :]
                        return acc

                    acc = lax.fori_loop(0, 7, row_body, acc)
                    sums_v[(g * K + j) * 2 + half, :] = acc + bias

        fire_group(0, 0)

        def outer(i, _):
            g0 = 2 * i
            fire_group(g0 + 1, 1)
            drain_group(0)
            sum_group(g0, 0)
            g2 = lax.rem(g0 + 2, GROUPS)
            fire_group(g2, 0)
            drain_group(1)
            sum_group(g0 + 1, 1)
            return 0

        lax.fori_loop(0, GROUPS // 2, outer, 0)
        drain_group(0)
        pltpu.sync_copy(sums_v, out_hbm.at[pl.ds(wid * BAGS_PER_W, BAGS_PER_W)])

    return bagsum


def kernel(text, emb_weight, fc1_w, fc1_b, fc2_w, fc2_b):
    et = emb_weight.T
    et_tail = lax.slice(et, (0, MAIN), (EMBED_DIM, VOCAB))
    w2t = jnp.pad(fc2_w.T, ((0, 0), (0, CPAD - NCLS)))
    b2p = jnp.pad(fc2_b, (0, CPAD - NCLS)).reshape(1, CPAD)
    p = _project(et, et_tail, fc1_w.T, w2t, fc1_b.reshape(1, 32), b2p)
    table16 = p.reshape(GROWS, CPAD)

    v = text.astype(jnp.int32)
    linrow = jnp.where(
        v < MAIN,
        (v & ~(STEP - 1)) | ((v & (R - 1)) << 3) | ((v >> 10) & (NK - 1)),
        MAIN + ((v - MAIN) << 3),
    )
    pairs = linrow.reshape(BATCH // 2, 2 * HIST)
    idx = jnp.pad(pairs, ((0, 0), (0, PAIR - 2 * HIST))).reshape(-1)

    sums = _make_bagsum()(idx, table16)
    return sums[:, :NCLS]
